# Initial kernel scaffold; baseline (speedup 1.0000x reference)
import jax
import jax.numpy as jnp
from jax.experimental import pallas as pl


def kernel(x, table):
    raise NotImplementedError("write your pallas kernel here")


# speedup vs baseline: 5.1079x; 5.1079x over previous
import functools
import math

import jax
import jax.numpy as jnp
from jax import lax
from jax.experimental import pallas as pl
from jax.experimental.pallas import tpu as pltpu
from jax.experimental.pallas import tpu_sc as plsc

D_MODEL = 128
SCALE = math.sqrt(D_MODEL)

NUM_CORES = 2
NUM_SUBCORES = 16
NW = NUM_CORES * NUM_SUBCORES
LANES = 16

CHUNK = 128
B_TOTAL = 4096 * 200
B_PER_W = B_TOTAL // NW
CHUNKS_PER_W = B_PER_W // CHUNK


@functools.partial(
    pl.kernel,
    mesh=plsc.VectorSubcoreMesh(core_axis_name="c", subcore_axis_name="s"),
    out_type=jax.ShapeDtypeStruct((B_TOTAL, D_MODEL), jnp.float32),
    scratch_types=[
        pltpu.VMEM((CHUNKS_PER_W, CHUNK), jnp.int32),
        pltpu.VMEM((CHUNK, D_MODEL), jnp.float32),
        pltpu.SemaphoreType.DMA,
    ],
)
def _emb_lookup(x_hbm, table_hbm, out_hbm, idx_v, rows_v, sem):
    wid = lax.axis_index("s") * NUM_CORES + lax.axis_index("c")
    base = wid * B_PER_W

    pltpu.sync_copy(x_hbm.at[wid], idx_v)

    def chunk_body(g, carry):
        pltpu.async_copy(table_hbm.at[idx_v.at[g]], rows_v, sem).wait()

        def row_body(r, carry2):
            for j in range(D_MODEL // LANES):
                sl = pl.ds(j * LANES, LANES)
                rows_v[r, sl] = rows_v[r, sl] * SCALE
            return carry2

        lax.fori_loop(0, CHUNK, row_body, 0, unroll=False)

        pltpu.sync_copy(rows_v, out_hbm.at[pl.ds(base + g * CHUNK, CHUNK)])
        return carry

    lax.fori_loop(0, CHUNKS_PER_W, chunk_body, 0, unroll=False)


def kernel(x, table):
    xf = x.reshape(NW, CHUNKS_PER_W, CHUNK).astype(jnp.int32)
    out = _emb_lookup(xf, table)
    return out.reshape(x.shape[0], x.shape[1], D_MODEL)


# speedup vs baseline: 8.2652x; 1.6181x over previous
import functools
import math

import jax
import jax.numpy as jnp
from jax import lax
from jax.experimental import pallas as pl
from jax.experimental.pallas import tpu as pltpu
from jax.experimental.pallas import tpu_sc as plsc

D_MODEL = 128
SCALE = math.sqrt(D_MODEL)

NUM_CORES = 2
NUM_SUBCORES = 16
NW = NUM_CORES * NUM_SUBCORES
LANES = 16

CHUNK = 128
B_TOTAL = 4096 * 200
B_PER_W = B_TOTAL // NW
CHUNKS_PER_W = B_PER_W // CHUNK
PAIRS = CHUNKS_PER_W // 2


@functools.partial(
    pl.kernel,
    mesh=plsc.VectorSubcoreMesh(core_axis_name="c", subcore_axis_name="s"),
    out_type=jax.ShapeDtypeStruct((B_TOTAL, D_MODEL), jnp.float32),
    scratch_types=[
        pltpu.VMEM((CHUNKS_PER_W, CHUNK), jnp.int32),
        pltpu.VMEM((CHUNK, D_MODEL), jnp.float32),
        pltpu.VMEM((CHUNK, D_MODEL), jnp.float32),
        pltpu.SemaphoreType.DMA,
        pltpu.SemaphoreType.DMA,
        pltpu.SemaphoreType.DMA,
        pltpu.SemaphoreType.DMA,
    ],
)
def _emb_lookup(x_hbm, table_hbm, out_hbm, idx_v, rows0, rows1,
                gs0, gs1, ss0, ss1):
    wid = lax.axis_index("s") * NUM_CORES + lax.axis_index("c")
    base = wid * B_PER_W

    pltpu.sync_copy(x_hbm.at[wid], idx_v)

    def gather_start(g, buf, sem):
        pltpu.async_copy(table_hbm.at[idx_v.at[g]], buf, sem)

    def gather_wait(g, buf, sem):
        pltpu.make_async_copy(table_hbm.at[idx_v.at[g]], buf, sem).wait()

    def scatter_start(g, buf, sem):
        pltpu.async_copy(buf, out_hbm.at[pl.ds(base + g * CHUNK, CHUNK)], sem)

    def scatter_wait(g, buf, sem):
        pltpu.make_async_copy(
            buf, out_hbm.at[pl.ds(base + g * CHUNK, CHUNK)], sem).wait()

    def scale(buf):
        def row_body(r, carry):
            for j in range(D_MODEL // LANES):
                sl = pl.ds(j * LANES, LANES)
                buf[r, sl] = buf[r, sl] * SCALE
            return carry
        lax.fori_loop(0, CHUNK, row_body, 0, unroll=2)

    gather_start(0, rows0, gs0)

    def pair_body(p, carry):
        g0 = 2 * p

        @pl.when(p > 0)
        def _():
            scatter_wait(g0 - 1, rows1, ss1)
        gather_start(g0 + 1, rows1, gs1)

        gather_wait(g0, rows0, gs0)
        scale(rows0)
        scatter_start(g0, rows0, ss0)

        gather_wait(g0 + 1, rows1, gs1)
        scale(rows1)
        scatter_start(g0 + 1, rows1, ss1)

        @pl.when(p < PAIRS - 1)
        def _():
            scatter_wait(g0, rows0, ss0)
            gather_start(g0 + 2, rows0, gs0)
        return carry

    lax.fori_loop(0, PAIRS, pair_body, 0, unroll=False)

    scatter_wait(2 * PAIRS - 2, rows0, ss0)
    scatter_wait(2 * PAIRS - 1, rows1, ss1)


def kernel(x, table):
    xf = x.reshape(NW, CHUNKS_PER_W, CHUNK).astype(jnp.int32)
    out = _emb_lookup(xf, table)
    return out.reshape(x.shape[0], x.shape[1], D_MODEL)


# speedup vs baseline: 9.1625x; 1.1086x over previous
import functools
import math

import jax
import jax.numpy as jnp
from jax import lax
from jax.experimental import pallas as pl
from jax.experimental.pallas import tpu as pltpu
from jax.experimental.pallas import tpu_sc as plsc

D_MODEL = 128
SCALE = math.sqrt(D_MODEL)

NUM_CORES = 2
NUM_SUBCORES = 16
NW = NUM_CORES * NUM_SUBCORES
LANES = 16

CHUNK = 128
B_TOTAL = 4096 * 200
B_PER_W = B_TOTAL // NW
CHUNKS_PER_W = B_PER_W // CHUNK
NBUF = 4
LOOKAHEAD = 2
QUADS = CHUNKS_PER_W // NBUF


@functools.partial(
    pl.kernel,
    mesh=plsc.VectorSubcoreMesh(core_axis_name="c", subcore_axis_name="s"),
    out_type=jax.ShapeDtypeStruct((B_TOTAL, D_MODEL), jnp.float32),
    scratch_types=[
        pltpu.VMEM((CHUNKS_PER_W, CHUNK), jnp.int32),
    ] + [pltpu.VMEM((CHUNK, D_MODEL), jnp.float32)] * NBUF
      + [pltpu.SemaphoreType.DMA] * (2 * NBUF),
)
def _emb_lookup(x_hbm, table_hbm, out_hbm, idx_v, b0, b1, b2, b3,
                g0, g1, g2, g3, s0, s1, s2, s3):
    bufs = (b0, b1, b2, b3)
    gsems = (g0, g1, g2, g3)
    ssems = (s0, s1, s2, s3)

    wid = lax.axis_index("s") * NUM_CORES + lax.axis_index("c")
    base = wid * B_PER_W

    pltpu.sync_copy(x_hbm.at[wid], idx_v)

    def gather_start(g, buf, sem):
        pltpu.async_copy(table_hbm.at[idx_v.at[g]], buf, sem)

    def gather_wait(g, buf, sem):
        pltpu.make_async_copy(table_hbm.at[idx_v.at[g]], buf, sem).wait()

    def scatter_start(g, buf, sem):
        pltpu.async_copy(buf, out_hbm.at[pl.ds(base + g * CHUNK, CHUNK)], sem)

    def scatter_wait(g, buf, sem):
        pltpu.make_async_copy(
            buf, out_hbm.at[pl.ds(base + g * CHUNK, CHUNK)], sem).wait()

    def scale(buf):
        def row_body(r, carry):
            for j in range(D_MODEL // LANES):
                sl = pl.ds(j * LANES, LANES)
                buf[r, sl] = buf[r, sl] * SCALE
            return carry
        lax.fori_loop(0, CHUNK, row_body, 0, unroll=2)

    for g in range(LOOKAHEAD):
        gather_start(g, bufs[g], gsems[g])

    def quad_body(p, carry):
        for b in range(NBUF):
            g = NBUF * p + b
            bb = (b + LOOKAHEAD) % NBUF

            @pl.when(g + LOOKAHEAD < CHUNKS_PER_W)
            def _():
                @pl.when(g >= NBUF - LOOKAHEAD)
                def _():
                    scatter_wait(g + LOOKAHEAD - NBUF, bufs[bb], ssems[bb])
                gather_start(g + LOOKAHEAD, bufs[bb], gsems[bb])

            gather_wait(g, bufs[b], gsems[b])
            scale(bufs[b])
            scatter_start(g, bufs[b], ssems[b])
        return carry

    lax.fori_loop(0, QUADS, quad_body, 0, unroll=False)

    for g in range(CHUNKS_PER_W - NBUF, CHUNKS_PER_W):
        b = g % NBUF
        scatter_wait(g, bufs[b], ssems[b])


def kernel(x, table):
    xf = x.reshape(NW, CHUNKS_PER_W, CHUNK).astype(jnp.int32)
    out = _emb_lookup(xf, table)
    return out.reshape(x.shape[0], x.shape[1], D_MODEL)
